# baseline (speedup 1.0000x reference)
import functools

import jax
import jax.numpy as jnp
from jax import lax
from jax.experimental import pallas as pl
from jax.experimental.pallas import tpu as pltpu
from jax.experimental.pallas import tpu_sc as plsc

MODEL_DIM = 2048
N_EXPERTS = 16
N_TOKENS = 8192
BLK = 1024
GRID = N_TOKENS // BLK

NS = 16
CHUNK = N_TOKENS // NS
NG = CHUNK // 16


def _gate_block(x_ref, wg_ref, gates_ref, idx_ref, laux_ref, cnt_ref, me_ref):
    pid = pl.program_id(0)

    @pl.when(pid == 0)
    def _init():
        cnt_ref[...] = jnp.zeros_like(cnt_ref)
        me_ref[...] = jnp.zeros_like(me_ref)

    w = wg_ref[...]
    logits = jax.lax.dot_general(
        w, x_ref[...], (((1,), (1,)), ((), ())),
        preferred_element_type=jnp.float32)

    maxv = jnp.max(logits, axis=0, keepdims=True)
    ex = jnp.exp(logits - maxv)
    s = jnp.sum(ex, axis=0, keepdims=True)
    gates_ref[...] = jnp.reshape(1.0 / s, (BLK,))

    eidx = jax.lax.broadcasted_iota(jnp.int32, (N_EXPERTS, BLK), 0)
    cand = jnp.where(logits == maxv, eidx, N_EXPERTS)
    idx = jnp.min(cand, axis=0, keepdims=True)
    idx_ref[...] = jnp.reshape(idx, (BLK,))

    mask = (eidx == idx).astype(jnp.float32)
    cnt_ref[...] = cnt_ref[...] + jnp.sum(mask, axis=1, keepdims=True)
    me_ref[...] = me_ref[...] + jnp.sum(ex / s, axis=1, keepdims=True)

    @pl.when(pid == GRID - 1)
    def _fini():
        val = jnp.sum(me_ref[...] * cnt_ref[...]) * (
            N_EXPERTS / (N_TOKENS * N_TOKENS))
        laux_ref[...] = jnp.full((1, 1), val, dtype=jnp.float32)


def _tc_gate(input, wg):
    out_shapes = (
        jax.ShapeDtypeStruct((N_TOKENS,), jnp.float32),
        jax.ShapeDtypeStruct((N_TOKENS,), jnp.int32),
        jax.ShapeDtypeStruct((1, 1), jnp.float32),
    )
    return pl.pallas_call(
        _gate_block,
        grid=(GRID,),
        in_specs=[
            pl.BlockSpec((BLK, MODEL_DIM), lambda i: (i, 0)),
            pl.BlockSpec((N_EXPERTS, MODEL_DIM), lambda i: (0, 0)),
        ],
        out_specs=(
            pl.BlockSpec((BLK,), lambda i: (i,)),
            pl.BlockSpec((BLK,), lambda i: (i,)),
            pl.BlockSpec((1, 1), lambda i: (0, 0)),
        ),
        out_shape=out_shapes,
        scratch_shapes=[
            pltpu.VMEM((N_EXPERTS, 1), jnp.float32),
            pltpu.VMEM((N_EXPERTS, 1), jnp.float32),
        ],
    )(input, wg)


def _sc_local_body(idx_hbm, loc_hbm, hist_hbm, idxv, locv, histv):
    s = lax.axis_index("s")
    base = s * CHUNK
    pltpu.sync_copy(idx_hbm.at[pl.ds(base, CHUNK)], idxv)
    lane = lax.iota(jnp.int32, 16)

    def body1(g, cnt):
        v = idxv[pl.ds(pl.multiple_of(g * 16, 16), 16)]
        bvec = jnp.take_along_axis(cnt, v, axis=0)
        rank = jnp.zeros((16,), jnp.int32)
        hist = jnp.zeros((16,), jnp.int32)
        for l in range(16):
            el = jnp.take_along_axis(v, jnp.full((16,), l, jnp.int32), axis=0)
            m = v == el
            mm = jnp.logical_and(m, lane < l)
            r = plsc.all_reduce_population_count(mm)
            rank = jnp.where(lane == l, r, rank)
            hist = hist + jnp.where(lane == el, 1, 0)
        locv[pl.ds(pl.multiple_of(g * 16, 16), 16)] = bvec + rank
        return cnt + hist

    cnt = lax.fori_loop(0, NG, body1, jnp.zeros((16,), jnp.int32))
    histv[...] = cnt
    pltpu.sync_copy(histv, hist_hbm.at[pl.ds(pl.multiple_of(s * 16, 16), 16)])
    pltpu.sync_copy(locv, loc_hbm.at[pl.ds(base, CHUNK)])


def _sc_offset_body(idx_hbm, locl_hbm, hist_hbm, loc_hbm, idxv, locv, hallv):
    s = lax.axis_index("s")
    base = s * CHUNK
    pltpu.sync_copy(idx_hbm.at[pl.ds(base, CHUNK)], idxv)
    pltpu.sync_copy(locl_hbm.at[pl.ds(base, CHUNK)], locv)
    pltpu.sync_copy(hist_hbm, hallv)
    off = jnp.zeros((16,), jnp.int32)
    for w in range(NS):
        gate = jnp.where(s > w, 1, 0)
        off = off + hallv[pl.ds(w * 16, 16)] * gate

    def body2(g, carry):
        v = idxv[pl.ds(pl.multiple_of(g * 16, 16), 16)]
        lv = locv[pl.ds(pl.multiple_of(g * 16, 16), 16)]
        locv[pl.ds(pl.multiple_of(g * 16, 16), 16)] = (
            lv + jnp.take_along_axis(off, v, axis=0))
        return carry

    lax.fori_loop(0, NG, body2, 0)
    pltpu.sync_copy(locv, loc_hbm.at[pl.ds(base, CHUNK)])


def _sc_loc(idx):
    mesh = plsc.VectorSubcoreMesh(core_axis_name="c", subcore_axis_name="s",
                                  num_cores=1, num_subcores=NS)
    f1 = pl.kernel(
        _sc_local_body,
        out_type=(jax.ShapeDtypeStruct((N_TOKENS,), jnp.int32),
                  jax.ShapeDtypeStruct((NS * 16,), jnp.int32)),
        mesh=mesh,
        scratch_types=[
            pltpu.VMEM((CHUNK,), jnp.int32),
            pltpu.VMEM((CHUNK,), jnp.int32),
            pltpu.VMEM((16,), jnp.int32),
        ],
        compiler_params=pltpu.CompilerParams(needs_layout_passes=False),
    )
    loc_local, hist = f1(idx)
    f2 = pl.kernel(
        _sc_offset_body,
        out_type=jax.ShapeDtypeStruct((N_TOKENS,), jnp.int32),
        mesh=mesh,
        scratch_types=[
            pltpu.VMEM((CHUNK,), jnp.int32),
            pltpu.VMEM((CHUNK,), jnp.int32),
            pltpu.VMEM((NS * 16,), jnp.int32),
        ],
        compiler_params=pltpu.CompilerParams(needs_layout_passes=False),
    )
    return f2(idx, loc_local, hist)


def kernel(input, wg):
    gates1_s, idx, laux = _tc_gate(input, wg)
    loc = _sc_loc(idx)
    return (laux[0, 0], gates1_s, idx, loc)


# speedup vs baseline: 1.1355x; 1.1355x over previous
import functools

import jax
import jax.numpy as jnp
from jax import lax
from jax.experimental import pallas as pl
from jax.experimental.pallas import tpu as pltpu
from jax.experimental.pallas import tpu_sc as plsc

MODEL_DIM = 2048
N_EXPERTS = 16
N_TOKENS = 8192
BLK = 1024
GRID = N_TOKENS // BLK

NS = 16
CHUNK = N_TOKENS // NS
NG = CHUNK // 16


def _gate_block(x_ref, wg_ref, gates_ref, idx_ref, laux_ref, hists_ref,
                cnt_ref, me_ref):
    pid = pl.program_id(0)

    @pl.when(pid == 0)
    def _init():
        cnt_ref[...] = jnp.zeros_like(cnt_ref)
        me_ref[...] = jnp.zeros_like(me_ref)

    w = wg_ref[...]
    logits = jax.lax.dot_general(
        w, x_ref[...], (((1,), (1,)), ((), ())),
        preferred_element_type=jnp.float32)

    maxv = jnp.max(logits, axis=0, keepdims=True)
    ex = jnp.exp(logits - maxv)
    s = jnp.sum(ex, axis=0, keepdims=True)
    gates_ref[...] = jnp.reshape(1.0 / s, (BLK,))

    eidx = jax.lax.broadcasted_iota(jnp.int32, (N_EXPERTS, BLK), 0)
    cand = jnp.where(logits == maxv, eidx, N_EXPERTS)
    idx = jnp.min(cand, axis=0, keepdims=True)
    idx_ref[...] = jnp.reshape(idx, (BLK,))

    mask = (eidx == idx).astype(jnp.float32)
    ci = jax.lax.broadcasted_iota(jnp.int32, (BLK // CHUNK, BLK), 1)
    ri = jax.lax.broadcasted_iota(jnp.int32, (BLK // CHUNK, BLK), 0)
    cmask = ((ci >> 9) == ri).astype(jnp.float32)
    hb = jax.lax.dot_general(
        cmask, mask, (((1,), (1,)), ((), ())),
        preferred_element_type=jnp.float32)
    hists_ref[...] = jnp.reshape(hb.astype(jnp.int32),
                                 (1, BLK // CHUNK, N_EXPERTS))
    cnt_ref[...] = cnt_ref[...] + jnp.sum(mask, axis=1, keepdims=True)
    me_ref[...] = me_ref[...] + jnp.sum(ex / s, axis=1, keepdims=True)

    @pl.when(pid == GRID - 1)
    def _fini():
        val = jnp.sum(me_ref[...] * cnt_ref[...]) * (
            N_EXPERTS / (N_TOKENS * N_TOKENS))
        laux_ref[...] = jnp.full((1, 1), val, dtype=jnp.float32)


def _tc_gate(input, wg):
    out_shapes = (
        jax.ShapeDtypeStruct((N_TOKENS,), jnp.float32),
        jax.ShapeDtypeStruct((N_TOKENS,), jnp.int32),
        jax.ShapeDtypeStruct((1, 1), jnp.float32),
        jax.ShapeDtypeStruct((GRID, BLK // CHUNK, N_EXPERTS), jnp.int32),
    )
    return pl.pallas_call(
        _gate_block,
        grid=(GRID,),
        in_specs=[
            pl.BlockSpec((BLK, MODEL_DIM), lambda i: (i, 0)),
            pl.BlockSpec((N_EXPERTS, MODEL_DIM), lambda i: (0, 0)),
        ],
        out_specs=(
            pl.BlockSpec((BLK,), lambda i: (i,)),
            pl.BlockSpec((BLK,), lambda i: (i,)),
            pl.BlockSpec((1, 1), lambda i: (0, 0)),
            pl.BlockSpec((1, BLK // CHUNK, N_EXPERTS), lambda i: (i, 0, 0)),
        ),
        out_shape=out_shapes,
        scratch_shapes=[
            pltpu.VMEM((N_EXPERTS, 1), jnp.float32),
            pltpu.VMEM((N_EXPERTS, 1), jnp.float32),
        ],
    )(input, wg)


def _sc_loc_body(idx_hbm, hist_hbm, loc_hbm, idxv, locv, hallv):
    s = lax.axis_index("s")
    base = s * CHUNK
    pltpu.sync_copy(idx_hbm.at[pl.ds(base, CHUNK)], idxv)
    pltpu.sync_copy(hist_hbm, hallv)
    lane = lax.iota(jnp.int32, 16)
    off = jnp.zeros((16,), jnp.int32)
    for w in range(NS):
        gate = jnp.where(s > w, 1, 0)
        off = off + hallv[pl.ds(w * 16, 16)] * gate

    def body1(g, cnt):
        v = idxv[pl.ds(pl.multiple_of(g * 16, 16), 16)]
        bvec = jnp.take_along_axis(cnt, v, axis=0)
        rank = jnp.zeros((16,), jnp.int32)
        hist = jnp.zeros((16,), jnp.int32)
        for l in range(16):
            el = jnp.take_along_axis(v, jnp.full((16,), l, jnp.int32), axis=0)
            m = v == el
            mm = jnp.logical_and(m, lane < l)
            r = plsc.all_reduce_population_count(mm)
            rank = jnp.where(lane == l, r, rank)
            hist = hist + jnp.where(lane == el, 1, 0)
        locv[pl.ds(pl.multiple_of(g * 16, 16), 16)] = bvec + rank
        return cnt + hist

    lax.fori_loop(0, NG, body1, off)
    pltpu.sync_copy(locv, loc_hbm.at[pl.ds(base, CHUNK)])


def _sc_loc(idx, hists):
    mesh = plsc.VectorSubcoreMesh(core_axis_name="c", subcore_axis_name="s",
                                  num_cores=1, num_subcores=NS)
    f = pl.kernel(
        _sc_loc_body,
        out_type=jax.ShapeDtypeStruct((N_TOKENS,), jnp.int32),
        mesh=mesh,
        scratch_types=[
            pltpu.VMEM((CHUNK,), jnp.int32),
            pltpu.VMEM((CHUNK,), jnp.int32),
            pltpu.VMEM((NS * 16,), jnp.int32),
        ],
        compiler_params=pltpu.CompilerParams(needs_layout_passes=False),
    )
    return f(idx, jnp.reshape(hists, (NS * 16,)))


def kernel(input, wg):
    gates1_s, idx, laux, hists = _tc_gate(input, wg)
    loc = _sc_loc(idx, hists)
    return (laux[0, 0], gates1_s, idx, loc)
